# Initial kernel scaffold; baseline (speedup 1.0000x reference)
import jax
import jax.numpy as jnp
from jax.experimental import pallas as pl


def kernel(x, edge_index, W, b):
    raise NotImplementedError("write your pallas kernel here")


# speedup vs baseline: 28.5021x; 28.5021x over previous
import functools

import jax
import jax.numpy as jnp
from jax import lax
from jax.experimental import pallas as pl
from jax.experimental.pallas import tpu as pltpu
from jax.experimental.pallas import tpu_sc as plsc

N = 10000
E = 320000
D_IN = 128
D_OUT = 4

NC = 2
NS = 16
NW = NC * NS
CHUNK = 128
CPW = -(-E // (NW * CHUNK))
E_PAD = NW * CPW * CHUNK
DUMMY = N
N_PAD = ((N + NS * 8 - 1) // (NS * 8)) * (NS * 8)
RPB = N_PAD // NS

_mesh = plsc.VectorSubcoreMesh(
    core_axis_name="c", subcore_axis_name="s", num_cores=NC, num_subcores=NS
)


@functools.partial(
    pl.kernel,
    out_type=jax.ShapeDtypeStruct((NC, N_PAD, D_OUT), jnp.float32),
    mesh=_mesh,
    scratch_types=[
        pltpu.VMEM((CPW, CHUNK), jnp.int32),
        pltpu.VMEM((CPW, CHUNK), jnp.int32),
        pltpu.VMEM((CHUNK, D_OUT), jnp.float32),
        pltpu.VMEM_SHARED((N_PAD, D_OUT), jnp.float32),
        pltpu.SemaphoreType.DMA,
    ],
    compiler_params=pltpu.CompilerParams(use_tc_tiling_on_sc=False),
)
def _prop(table_hbm, init_hbm, rows_hbm, cols_hbm, out_hbm,
          rowv, colv, msgs, acc, sem):
    c = lax.axis_index("c")
    s = lax.axis_index("s")
    w = c * NS + s
    pltpu.sync_copy(rows_hbm.at[w], rowv)
    pltpu.sync_copy(cols_hbm.at[w], colv)
    pltpu.sync_copy(init_hbm.at[pl.ds(s * RPB, RPB)],
                    acc.at[pl.ds(s * RPB, RPB)])
    plsc.subcore_barrier()

    def body(j, carry):
        pltpu.async_copy(table_hbm.at[rowv.at[j]], msgs, sem).wait()
        pltpu.sync_copy(msgs, acc.at[colv.at[j]], add=True)
        return carry

    lax.fori_loop(0, CPW, body, 0)
    plsc.subcore_barrier()
    pltpu.sync_copy(acc.at[pl.ds(s * RPB, RPB)],
                    out_hbm.at[c, pl.ds(s * RPB, RPB)])


def _tc_prep(x_ref, w_ref, degp_ref, u_ref, dis_ref, dinv_ref):
    z = jnp.dot(x_ref[...], w_ref[...], preferred_element_type=jnp.float32)
    deg = degp_ref[0] + degp_ref[1] + 1.0
    dis = lax.rsqrt(deg)
    u_ref[...] = dis * z
    dis_ref[...] = dis
    dinv_ref[...] = 1.0 / deg


def _tc_mid(p_ref, u_ref, dinv_ref, w_ref):
    v = p_ref[0] + p_ref[1] - u_ref[...]
    w_ref[...] = v * dinv_ref[...]


def _tc_final(q_ref, w_ref, dis_ref, b_ref, out_ref):
    t = q_ref[0] + q_ref[1] - w_ref[...]
    o = dis_ref[...] * t + b_ref[...]
    m = jnp.max(o, axis=1, keepdims=True)
    e = jnp.exp(o - m)
    lse = jnp.log(jnp.sum(e, axis=1, keepdims=True))
    out_ref[...] = o - m - lse


def kernel(x, edge_index, W, b):
    f32 = jnp.float32
    rows = edge_index[0]
    cols = edge_index[1]
    pad = E_PAD - E
    rows3 = jnp.concatenate(
        [rows, jnp.zeros((pad,), jnp.int32)]).reshape(NW, CPW, CHUNK)
    cols3 = jnp.concatenate(
        [cols, jnp.full((pad,), DUMMY, jnp.int32)]).reshape(NW, CPW, CHUNK)
    x_pad = jnp.pad(x, ((0, N_PAD - N), (0, 0)))
    ones_t = jnp.ones((N_PAD, D_OUT), f32)
    zeros_t = jnp.zeros((N_PAD, D_OUT), f32)

    degp = _prop(ones_t, zeros_t, rows3, cols3)

    u, dis, dinv = pl.pallas_call(
        _tc_prep,
        out_shape=[jax.ShapeDtypeStruct((N_PAD, D_OUT), f32)] * 3,
    )(x_pad, W, degp)

    p = _prop(u, u, rows3, cols3)

    w = pl.pallas_call(
        _tc_mid,
        out_shape=jax.ShapeDtypeStruct((N_PAD, D_OUT), f32),
    )(p, u, dinv)

    q = _prop(w, w, rows3, cols3)

    out = pl.pallas_call(
        _tc_final,
        out_shape=jax.ShapeDtypeStruct((N_PAD, D_OUT), f32),
    )(q, w, dis, b)

    return out[:N]


# speedup vs baseline: 33.9033x; 1.1895x over previous
import functools

import jax
import jax.numpy as jnp
from jax import lax
from jax.experimental import pallas as pl
from jax.experimental.pallas import tpu as pltpu
from jax.experimental.pallas import tpu_sc as plsc

N = 10000
E = 320000
D_IN = 128
D_OUT = 4

NC = 2
NS = 16
NW = NC * NS
CHUNK = 128
NBUF = 8
CPW = 80
NGRP = CPW // NBUF
E_PAD = NW * CPW * CHUNK
DUMMY = N
N_PAD = ((N + NS * 8 - 1) // (NS * 8)) * (NS * 8)
RPB = N_PAD // NS

_mesh = plsc.VectorSubcoreMesh(
    core_axis_name="c", subcore_axis_name="s", num_cores=NC, num_subcores=NS
)


@functools.partial(
    pl.kernel,
    out_type=jax.ShapeDtypeStruct((NC, N_PAD, D_OUT), jnp.float32),
    mesh=_mesh,
    scratch_types=[
        pltpu.VMEM((CPW, CHUNK), jnp.int32),
        pltpu.VMEM((CPW, CHUNK), jnp.int32),
        pltpu.VMEM((NBUF, CHUNK, D_OUT), jnp.float32),
        pltpu.VMEM_SHARED((N_PAD, D_OUT), jnp.float32),
        pltpu.SemaphoreType.DMA,
        pltpu.SemaphoreType.DMA,
    ],
    compiler_params=pltpu.CompilerParams(use_tc_tiling_on_sc=False),
)
def _prop(table_hbm, init_hbm, rows_hbm, cols_hbm, out_hbm,
          rowv, colv, msgs, acc, gsem, ssem):
    c = lax.axis_index("c")
    s = lax.axis_index("s")
    w = c * NS + s
    pltpu.sync_copy(rows_hbm.at[w], rowv)
    pltpu.sync_copy(cols_hbm.at[w], colv)
    pltpu.sync_copy(init_hbm.at[pl.ds(s * RPB, RPB)],
                    acc.at[pl.ds(s * RPB, RPB)])
    plsc.subcore_barrier()

    def group(g, carry):
        gds = [pltpu.async_copy(table_hbm.at[rowv.at[g * NBUF + b]],
                                msgs.at[b], gsem)
               for b in range(NBUF)]
        sds = []
        for b in range(NBUF):
            gds[b].wait()
            sds.append(pltpu.async_copy(msgs.at[b],
                                        acc.at[colv.at[g * NBUF + b]],
                                        ssem, add=True))
        for d in sds:
            d.wait()
        return carry

    lax.fori_loop(0, NGRP, group, 0)
    plsc.subcore_barrier()
    pltpu.sync_copy(acc.at[pl.ds(s * RPB, RPB)],
                    out_hbm.at[c, pl.ds(s * RPB, RPB)])


def _tc_prep(x_ref, w_ref, degp_ref, u_ref, dis_ref, dinv_ref):
    z = jnp.dot(x_ref[...], w_ref[...], preferred_element_type=jnp.float32)
    deg = degp_ref[0] + degp_ref[1] + 1.0
    dis = lax.rsqrt(deg)
    u_ref[...] = dis * z
    dis_ref[...] = dis
    dinv_ref[...] = 1.0 / deg


def _tc_mid(p_ref, u_ref, dinv_ref, w_ref):
    v = p_ref[0] + p_ref[1] - u_ref[...]
    w_ref[...] = v * dinv_ref[...]


def _tc_final(q_ref, w_ref, dis_ref, b_ref, out_ref):
    t = q_ref[0] + q_ref[1] - w_ref[...]
    o = dis_ref[...] * t + b_ref[...]
    m = jnp.max(o, axis=1, keepdims=True)
    e = jnp.exp(o - m)
    lse = jnp.log(jnp.sum(e, axis=1, keepdims=True))
    out_ref[...] = o - m - lse


def kernel(x, edge_index, W, b):
    f32 = jnp.float32
    rows = edge_index[0]
    cols = edge_index[1]
    pad = E_PAD - E
    rows3 = jnp.concatenate(
        [rows, jnp.zeros((pad,), jnp.int32)]).reshape(NW, CPW, CHUNK)
    cols3 = jnp.concatenate(
        [cols, jnp.full((pad,), DUMMY, jnp.int32)]).reshape(NW, CPW, CHUNK)
    x_pad = jnp.pad(x, ((0, N_PAD - N), (0, 0)))
    ones_t = jnp.ones((N_PAD, D_OUT), f32)
    zeros_t = jnp.zeros((N_PAD, D_OUT), f32)

    degp = _prop(ones_t, zeros_t, rows3, cols3)

    u, dis, dinv = pl.pallas_call(
        _tc_prep,
        out_shape=[jax.ShapeDtypeStruct((N_PAD, D_OUT), f32)] * 3,
    )(x_pad, W, degp)

    p = _prop(u, u, rows3, cols3)

    w = pl.pallas_call(
        _tc_mid,
        out_shape=jax.ShapeDtypeStruct((N_PAD, D_OUT), f32),
    )(p, u, dinv)

    q = _prop(w, w, rows3, cols3)

    out = pl.pallas_call(
        _tc_final,
        out_shape=jax.ShapeDtypeStruct((N_PAD, D_OUT), f32),
    )(q, w, dis, b)

    return out[:N]


# speedup vs baseline: 44.2480x; 1.3051x over previous
import functools

import jax
import jax.numpy as jnp
from jax import lax
from jax.experimental import pallas as pl
from jax.experimental.pallas import tpu as pltpu
from jax.experimental.pallas import tpu_sc as plsc

N = 10000
E = 320000
D_IN = 128
D_OUT = 4

NC = 2
NS = 16
NW = NC * NS
CHUNK = 128
NBUF = 8
CPW = 80
NGRP = CPW // NBUF
E_PAD = NW * CPW * CHUNK
DUMMY = N
N_PAD = ((N + NS * 8 - 1) // (NS * 8)) * (NS * 8)
RPB = N_PAD // NS

_mesh = plsc.VectorSubcoreMesh(
    core_axis_name="c", subcore_axis_name="s", num_cores=NC, num_subcores=NS
)


@functools.partial(
    pl.kernel,
    out_type=jax.ShapeDtypeStruct((NC, N_PAD, D_OUT), jnp.float32),
    mesh=_mesh,
    scratch_types=[
        pltpu.VMEM((CPW, CHUNK), jnp.int32),
        pltpu.VMEM((CPW, CHUNK), jnp.int32),
        pltpu.VMEM((NBUF, CHUNK, D_OUT), jnp.float32),
        pltpu.VMEM((N_PAD, D_OUT), jnp.float32),
        pltpu.VMEM_SHARED((N_PAD, D_OUT), jnp.float32),
        pltpu.SemaphoreType.DMA,
        pltpu.SemaphoreType.DMA,
    ],
    compiler_params=pltpu.CompilerParams(
        use_tc_tiling_on_sc=False, needs_layout_passes=False),
)
def _prop(table_hbm, init_hbm, rows_hbm, cols_hbm, out_hbm,
          rowv, colv, msgs, tbl, acc, gsem, ssem):
    c = lax.axis_index("c")
    s = lax.axis_index("s")
    w = c * NS + s
    pltpu.sync_copy(rows_hbm.at[w], rowv)
    pltpu.sync_copy(cols_hbm.at[w], colv)
    pltpu.sync_copy(table_hbm, tbl)
    pltpu.sync_copy(init_hbm.at[pl.ds(s * RPB, RPB)],
                    acc.at[pl.ds(s * RPB, RPB)])
    plsc.subcore_barrier()

    lanes = lax.iota(jnp.int32, 16)

    def group(g, carry):
        sds = []
        for b in range(NBUF):
            j = g * NBUF + b
            for i in range(CHUNK // 16):
                rows16 = rowv[j, pl.ds(i * 16, 16)]
                pos = lanes + (i * 16)
                for d in range(D_OUT):
                    dd = jnp.full((16,), d, jnp.int32)
                    v = plsc.load_gather(tbl, [rows16, dd])
                    plsc.store_scatter(msgs.at[b], [pos, dd], v)
            sds.append(pltpu.async_copy(msgs.at[b],
                                        acc.at[colv.at[j]],
                                        ssem, add=True))
        for dsc in sds:
            dsc.wait()
        return carry

    lax.fori_loop(0, NGRP, group, 0)
    plsc.subcore_barrier()
    pltpu.sync_copy(acc.at[pl.ds(s * RPB, RPB)],
                    out_hbm.at[c, pl.ds(s * RPB, RPB)])


def _tc_prep(x_ref, w_ref, degp_ref, u_ref, dis_ref, dinv_ref):
    z = jnp.dot(x_ref[...], w_ref[...], preferred_element_type=jnp.float32)
    deg = degp_ref[0] + degp_ref[1] + 1.0
    dis = lax.rsqrt(deg)
    u_ref[...] = dis * z
    dis_ref[...] = dis
    dinv_ref[...] = 1.0 / deg


def _tc_mid(p_ref, u_ref, dinv_ref, w_ref):
    v = p_ref[0] + p_ref[1] - u_ref[...]
    w_ref[...] = v * dinv_ref[...]


def _tc_final(q_ref, w_ref, dis_ref, b_ref, out_ref):
    t = q_ref[0] + q_ref[1] - w_ref[...]
    o = dis_ref[...] * t + b_ref[...]
    m = jnp.max(o, axis=1, keepdims=True)
    e = jnp.exp(o - m)
    lse = jnp.log(jnp.sum(e, axis=1, keepdims=True))
    out_ref[...] = o - m - lse


def kernel(x, edge_index, W, b):
    f32 = jnp.float32
    rows = edge_index[0]
    cols = edge_index[1]
    pad = E_PAD - E
    rows3 = jnp.concatenate(
        [rows, jnp.zeros((pad,), jnp.int32)]).reshape(NW, CPW, CHUNK)
    cols3 = jnp.concatenate(
        [cols, jnp.full((pad,), DUMMY, jnp.int32)]).reshape(NW, CPW, CHUNK)
    x_pad = jnp.pad(x, ((0, N_PAD - N), (0, 0)))
    ones_t = jnp.ones((N_PAD, D_OUT), f32)
    zeros_t = jnp.zeros((N_PAD, D_OUT), f32)

    degp = _prop(ones_t, zeros_t, rows3, cols3)

    u, dis, dinv = pl.pallas_call(
        _tc_prep,
        out_shape=[jax.ShapeDtypeStruct((N_PAD, D_OUT), f32)] * 3,
    )(x_pad, W, degp)

    p = _prop(u, u, rows3, cols3)

    w = pl.pallas_call(
        _tc_mid,
        out_shape=jax.ShapeDtypeStruct((N_PAD, D_OUT), f32),
    )(p, u, dinv)

    q = _prop(w, w, rows3, cols3)

    out = pl.pallas_call(
        _tc_final,
        out_shape=jax.ShapeDtypeStruct((N_PAD, D_OUT), f32),
    )(q, w, dis, b)

    return out[:N]
